# Initial kernel scaffold; baseline (speedup 1.0000x reference)
import jax
import jax.numpy as jnp
from jax.experimental import pallas as pl


def kernel(x, pos_emb):
    raise NotImplementedError("write your pallas kernel here")


# speedup vs baseline: 2.9946x; 2.9946x over previous
import jax
import jax.numpy as jnp
from jax.experimental import pallas as pl


def _copy_kernel(src_ref, out_ref):
    out_ref[...] = src_ref[...]


def kernel(x, pos_emb):
    seq_len = x.shape[1]
    rows, cols = pos_emb.shape
    block_rows = 1024
    grid = (seq_len // block_rows,)
    return pl.pallas_call(
        _copy_kernel,
        grid=grid,
        in_specs=[pl.BlockSpec((block_rows, cols), lambda i: (i, 0))],
        out_specs=pl.BlockSpec((block_rows, cols), lambda i: (i, 0)),
        out_shape=jax.ShapeDtypeStruct((seq_len, cols), pos_emb.dtype),
    )(pos_emb)
